# baseline (speedup 1.0000x reference)
import jax
import jax.numpy as jnp
from jax.experimental import pallas as pl
from jax.experimental.pallas import tpu as pltpu

_N = 256
_H = 256
_F = 2 * _H
_RB = 32

_DEFAULT = jax.lax.Precision.DEFAULT


def _mmT(x, w, precision=_DEFAULT):
    return jax.lax.dot_general(
        x, w, (((1,), (1,)), ((), ())),
        preferred_element_type=jnp.float32, precision=precision)


def _elu(x):
    return jnp.where(x > 0, x, jnp.exp(x) - 1.0)


def _decoder_body(edges_ref, hidden_ref, inputs_ref,
                  W1_ref, b1_ref, W2_ref, b2_ref,
                  Whr_ref, Whi_ref, Whh_ref,
                  Wir_ref, bir_ref, Wii_ref, bii_ref, Win_ref, binn_ref,
                  Wo1_ref, bo1_ref, Wo2_ref, bo2_ref, Wo3_ref, bo3_ref,
                  pred_ref, hidden_out_ref,
                  A_ref, B_ref, w_ref, S_ref, Q_ref):
    f32 = jnp.float32
    bf16 = jnp.bfloat16
    log2e = 1.4426950408889634
    ln2 = 0.6931471805599453
    hidden = hidden_ref[...]
    W1 = W1_ref[...]
    dflt = jax.lax.Precision.DEFAULT
    h2 = hidden * log2e
    A_ref[...] = _mmT(h2, W1[:, :_H], dflt).astype(bf16)
    B_ref[...] = _mmT(h2, W1[:, _H:], dflt).astype(bf16)

    e = edges_ref[...]
    m1 = jnp.maximum(e, jnp.concatenate([e[:, 1:], e[:, :1]], axis=1))
    m2 = jnp.maximum(m1, jnp.concatenate([m1[:, 2:], m1[:, :2]], axis=1))
    pr = jax.lax.broadcasted_iota(jnp.int32, (4 * (_N - 1), _N - 1), 0)
    pc = jax.lax.broadcasted_iota(jnp.int32, (4 * (_N - 1), _N - 1), 1)
    psel = (pr == 4 * pc).astype(f32)
    wmax = jax.lax.dot_general(m2, psel, (((1,), (0,)), ((), ())),
                               preferred_element_type=f32)
    zcol = jnp.zeros((_N, 1), f32)
    u = jnp.concatenate([wmax, zcol], axis=1)
    v = jnp.concatenate([zcol, wmax], axis=1)
    row = jax.lax.broadcasted_iota(jnp.int32, (_N, _N), 0)
    col = jax.lax.broadcasted_iota(jnp.int32, (_N, _N), 1)
    w_ref[...] = jnp.where(col < row, u, jnp.where(col > row, v, 0.0)
                           ).astype(bf16)

    b1 = b1_ref[...]
    b1_lo = (b1 * log2e).astype(bf16)
    S_ref[...] = jnp.zeros((_N, _F), f32)
    Q_ref[...] = jnp.zeros((_N, _F), f32)

    def _tree(t):
        s16 = t[0:16] + t[16:32]
        s8 = s16[0:8] + s16[8:16]
        s4 = s8[0:4] + s8[4:8]
        s2 = s4[0:2] + s4[2:4]
        return (s2[0] + s2[1]).astype(f32)

    def step(rb, carry):
        r0 = rb * _RB
        w_blk = w_ref[pl.ds(r0, _RB), :]
        b_blk = B_ref[pl.ds(r0, _RB), :]
        x = (w_blk[:, :, None] * (A_ref[...][None, :, :] + b_blk[:, None, :])
             + b1_lo[None, :, :])
        xn = jnp.minimum(x, 0)
        S_ref[...] += _tree(xn)
        Q_ref[...] += _tree(jnp.exp2(xn))
        return carry

    jax.lax.fori_loop(0, _N // _RB, step, 0)

    wmat = w_ref[...]
    sw = jnp.sum(wmat.astype(f32), axis=0).reshape(1, _N)
    wtB = jax.lax.dot_general(
        wmat, B_ref[...], (((0,), (0,)), ((), ())),
        preferred_element_type=f32)
    xlin = (A_ref[...].astype(f32) * sw.reshape(_N, 1)
            + wtB + _N * (b1 * log2e))
    S = (xlin - S_ref[...]) * ln2 + Q_ref[...] - (_N + _elu(b1))
    agg = _mmT(S, W2_ref[...]) * (1.0 / (_N - 1)) + b2_ref[...]

    inp = inputs_ref[...]
    r = jax.nn.sigmoid(_mmT(inp, Wir_ref[...]) + bir_ref[...]
                       + _mmT(agg, Whr_ref[...]))
    i = jax.nn.sigmoid(_mmT(inp, Wii_ref[...]) + bii_ref[...]
                       + _mmT(agg, Whi_ref[...]))
    n = jnp.tanh(_mmT(inp, Win_ref[...]) + binn_ref[...]
                 + r * _mmT(agg, Whh_ref[...]))
    hidden_new = (1.0 - i) * n + i * hidden

    p = jax.nn.relu(_mmT(hidden_new, Wo1_ref[...]) + bo1_ref[...])
    p = jax.nn.relu(_mmT(p, Wo2_ref[...]) + bo2_ref[...])
    p = _mmT(p, Wo3_ref[...]) + bo3_ref[...]
    pred_ref[...] = inp + p
    hidden_out_ref[...] = hidden_new


def kernel(inputs, hidden, edges, edge_index, W1, b1, W2, b2,
           Whr, Whi, Whh, Wir, bir, Wii, bii, Win, binn,
           Wo1, bo1, Wo2, bo2, Wo3, bo3):
    del edge_index
    f32 = jnp.float32
    edges_r = edges.reshape(_N, (_N - 1) * 4)
    row2 = lambda b: b.reshape(1, -1).astype(f32)
    out = pl.pallas_call(
        _decoder_body,
        out_shape=(jax.ShapeDtypeStruct(inputs.shape, f32),
                   jax.ShapeDtypeStruct(hidden.shape, f32)),
        scratch_shapes=[
            pltpu.VMEM((_N, _F), jnp.bfloat16),
            pltpu.VMEM((_N, _F), jnp.bfloat16),
            pltpu.VMEM((_N, _N), jnp.bfloat16),
            pltpu.VMEM((_N, _F), f32),
            pltpu.VMEM((_N, _F), f32),
        ],
    )(edges_r, hidden, inputs,
      W1, row2(b1), W2, row2(b2),
      Whr, Whi, Whh,
      Wir, row2(bir), Wii, row2(bii), Win, row2(binn),
      Wo1, row2(bo1), Wo2, row2(bo2), Wo3, row2(bo3))
    return out


# speedup vs baseline: 1.8715x; 1.8715x over previous
import jax
import jax.numpy as jnp
from jax.experimental import pallas as pl
from jax.experimental.pallas import tpu as pltpu

_N = 256
_H = 256
_F = 2 * _H
_RB = 32

_DEFAULT = jax.lax.Precision.DEFAULT


def _mmT(x, w, precision=_DEFAULT):
    return jax.lax.dot_general(
        x, w, (((1,), (1,)), ((), ())),
        preferred_element_type=jnp.float32, precision=precision)


def _elu(x):
    return jnp.where(x > 0, x, jnp.exp(x) - 1.0)


def _decoder_body(edges_ref, hidden_ref, inputs_ref,
                  W1_ref, b1_ref, W2_ref, b2_ref,
                  Whr_ref, Whi_ref, Whh_ref,
                  Wir_ref, bir_ref, Wii_ref, bii_ref, Win_ref, binn_ref,
                  Wo1_ref, bo1_ref, Wo2_ref, bo2_ref, Wo3_ref, bo3_ref,
                  pred_ref, hidden_out_ref,
                  A_ref, B_ref, w_ref, S_ref, Q_ref):
    f32 = jnp.float32
    bf16 = jnp.bfloat16
    log2e = 1.4426950408889634
    ln2 = 0.6931471805599453
    hidden = hidden_ref[...]
    W1 = W1_ref[...]
    dflt = jax.lax.Precision.DEFAULT
    h2 = hidden * log2e
    A_ref[...] = _mmT(h2, W1[:, :_H], dflt).astype(bf16)
    B_ref[...] = _mmT(h2, W1[:, _H:], dflt).astype(bf16)

    e4 = edges_ref[...]
    wmax = jnp.maximum(jnp.maximum(e4[0], e4[1]), jnp.maximum(e4[2], e4[3]))
    zcol = jnp.zeros((_N, 1), f32)
    u = jnp.concatenate([wmax, zcol], axis=1)
    v = jnp.concatenate([zcol, wmax], axis=1)
    row = jax.lax.broadcasted_iota(jnp.int32, (_N, _N), 0)
    col = jax.lax.broadcasted_iota(jnp.int32, (_N, _N), 1)
    w_ref[...] = jnp.where(col < row, u, jnp.where(col > row, v, 0.0)
                           ).astype(bf16)

    b1 = b1_ref[...]
    b1_lo = (b1 * log2e).astype(bf16)
    S_ref[...] = jnp.zeros((_N, _F), f32)
    Q_ref[...] = jnp.zeros((_N, _F), f32)

    def _tree(t):
        s16 = t[0:16] + t[16:32]
        s8 = s16[0:8] + s16[8:16]
        s4 = s8[0:4] + s8[4:8]
        s2 = s4[0:2] + s4[2:4]
        return (s2[0] + s2[1]).astype(f32)

    def step(rb, carry):
        r0 = rb * _RB
        w_blk = w_ref[pl.ds(r0, _RB), :]
        b_blk = B_ref[pl.ds(r0, _RB), :]
        x = (w_blk[:, :, None] * (A_ref[...][None, :, :] + b_blk[:, None, :])
             + b1_lo[None, :, :])
        xn = jnp.minimum(x, 0)
        S_ref[...] += _tree(xn)
        Q_ref[...] += _tree(jnp.exp2(xn))
        return carry

    jax.lax.fori_loop(0, _N // _RB, step, 0)

    wmat = w_ref[...]
    sw = jnp.sum(wmat.astype(f32), axis=0).reshape(1, _N)
    wtB = jax.lax.dot_general(
        wmat, B_ref[...], (((0,), (0,)), ((), ())),
        preferred_element_type=f32)
    xlin = (A_ref[...].astype(f32) * sw.reshape(_N, 1)
            + wtB + _N * (b1 * log2e))
    S = (xlin - S_ref[...]) * ln2 + Q_ref[...] - (_N + _elu(b1))
    agg = _mmT(S, W2_ref[...]) * (1.0 / (_N - 1)) + b2_ref[...]

    inp = inputs_ref[...]
    r = jax.nn.sigmoid(_mmT(inp, Wir_ref[...]) + bir_ref[...]
                       + _mmT(agg, Whr_ref[...]))
    i = jax.nn.sigmoid(_mmT(inp, Wii_ref[...]) + bii_ref[...]
                       + _mmT(agg, Whi_ref[...]))
    n = jnp.tanh(_mmT(inp, Win_ref[...]) + binn_ref[...]
                 + r * _mmT(agg, Whh_ref[...]))
    hidden_new = (1.0 - i) * n + i * hidden

    p = jax.nn.relu(_mmT(hidden_new, Wo1_ref[...]) + bo1_ref[...])
    p = jax.nn.relu(_mmT(p, Wo2_ref[...]) + bo2_ref[...])
    p = _mmT(p, Wo3_ref[...]) + bo3_ref[...]
    pred_ref[...] = inp + p
    hidden_out_ref[...] = hidden_new


def kernel(inputs, hidden, edges, edge_index, W1, b1, W2, b2,
           Whr, Whi, Whh, Wir, bir, Wii, bii, Win, binn,
           Wo1, bo1, Wo2, bo2, Wo3, bo3):
    del edge_index
    f32 = jnp.float32
    edges_t = jnp.transpose(edges.reshape(_N, _N - 1, 4), (2, 0, 1))
    row2 = lambda b: b.reshape(1, -1).astype(f32)
    out = pl.pallas_call(
        _decoder_body,
        out_shape=(jax.ShapeDtypeStruct(inputs.shape, f32),
                   jax.ShapeDtypeStruct(hidden.shape, f32)),
        scratch_shapes=[
            pltpu.VMEM((_N, _F), jnp.bfloat16),
            pltpu.VMEM((_N, _F), jnp.bfloat16),
            pltpu.VMEM((_N, _N), jnp.bfloat16),
            pltpu.VMEM((_N, _F), f32),
            pltpu.VMEM((_N, _F), f32),
        ],
    )(edges_t, hidden, inputs,
      W1, row2(b1), W2, row2(b2),
      Whr, Whi, Whh,
      Wir, row2(bir), Wii, row2(bii), Win, row2(binn),
      Wo1, row2(bo1), Wo2, row2(bo2), Wo3, row2(bo3))
    return out


# speedup vs baseline: 1.9000x; 1.0152x over previous
import jax
import jax.numpy as jnp
from jax.experimental import pallas as pl
from jax.experimental.pallas import tpu as pltpu

_N = 256
_H = 256
_F = 2 * _H
_RB = 64

_DEFAULT = jax.lax.Precision.DEFAULT


def _mmT(x, w, precision=_DEFAULT):
    return jax.lax.dot_general(
        x, w, (((1,), (1,)), ((), ())),
        preferred_element_type=jnp.float32, precision=precision)


def _elu(x):
    return jnp.where(x > 0, x, jnp.exp(x) - 1.0)


def _decoder_body(edges_ref, hidden_ref, inputs_ref,
                  W1_ref, b1_ref, W2_ref, b2_ref,
                  Whr_ref, Whi_ref, Whh_ref,
                  Wir_ref, bir_ref, Wii_ref, bii_ref, Win_ref, binn_ref,
                  Wo1_ref, bo1_ref, Wo2_ref, bo2_ref, Wo3_ref, bo3_ref,
                  pred_ref, hidden_out_ref,
                  A_ref, B_ref, w_ref, S_ref, Q_ref):
    f32 = jnp.float32
    bf16 = jnp.bfloat16
    log2e = 1.4426950408889634
    ln2 = 0.6931471805599453
    hidden = hidden_ref[...]
    W1 = W1_ref[...]
    dflt = jax.lax.Precision.DEFAULT
    h2 = hidden * log2e
    A_ref[...] = _mmT(h2, W1[:, :_H], dflt).astype(bf16)
    B_ref[...] = _mmT(h2, W1[:, _H:], dflt).astype(bf16)

    e4 = edges_ref[...]
    wmax = jnp.maximum(jnp.maximum(e4[0], e4[1]), jnp.maximum(e4[2], e4[3]))
    zcol = jnp.zeros((_N, 1), f32)
    u = jnp.concatenate([wmax, zcol], axis=1)
    v = jnp.concatenate([zcol, wmax], axis=1)
    row = jax.lax.broadcasted_iota(jnp.int32, (_N, _N), 0)
    col = jax.lax.broadcasted_iota(jnp.int32, (_N, _N), 1)
    w_ref[...] = jnp.where(col < row, u, jnp.where(col > row, v, 0.0)
                           ).astype(bf16)

    b1 = b1_ref[...]
    b1_lo = (b1 * log2e).astype(bf16)
    S_ref[...] = jnp.zeros((_N, _F), f32)
    Q_ref[...] = jnp.zeros((_N, _F), f32)

    def _tree(t):
        s32 = t[0:32] + t[32:64]
        s16 = s32[0:16] + s32[16:32]
        s8 = s16[0:8] + s16[8:16]
        s4 = s8[0:4] + s8[4:8]
        s2 = s4[0:2] + s4[2:4]
        return (s2[0] + s2[1]).astype(f32)

    def step(rb, carry):
        r0 = rb * _RB
        w_blk = w_ref[pl.ds(r0, _RB), :]
        b_blk = B_ref[pl.ds(r0, _RB), :]
        x = (w_blk[:, :, None] * (A_ref[...][None, :, :] + b_blk[:, None, :])
             + b1_lo[None, :, :])
        xn = jnp.minimum(x, 0)
        S_ref[...] += _tree(xn)
        Q_ref[...] += _tree(jnp.exp2(xn))
        return carry

    jax.lax.fori_loop(0, _N // _RB, step, 0)

    wmat = w_ref[...]
    sw = jnp.sum(wmat.astype(f32), axis=0).reshape(1, _N)
    wtB = jax.lax.dot_general(
        wmat, B_ref[...], (((0,), (0,)), ((), ())),
        preferred_element_type=f32)
    xlin = (A_ref[...].astype(f32) * sw.reshape(_N, 1)
            + wtB + _N * (b1 * log2e))
    S = (xlin - S_ref[...]) * ln2 + Q_ref[...] - (_N + _elu(b1))
    agg = _mmT(S, W2_ref[...]) * (1.0 / (_N - 1)) + b2_ref[...]

    inp = inputs_ref[...]
    r = jax.nn.sigmoid(_mmT(inp, Wir_ref[...]) + bir_ref[...]
                       + _mmT(agg, Whr_ref[...]))
    i = jax.nn.sigmoid(_mmT(inp, Wii_ref[...]) + bii_ref[...]
                       + _mmT(agg, Whi_ref[...]))
    n = jnp.tanh(_mmT(inp, Win_ref[...]) + binn_ref[...]
                 + r * _mmT(agg, Whh_ref[...]))
    hidden_new = (1.0 - i) * n + i * hidden

    p = jax.nn.relu(_mmT(hidden_new, Wo1_ref[...]) + bo1_ref[...])
    p = jax.nn.relu(_mmT(p, Wo2_ref[...]) + bo2_ref[...])
    p = _mmT(p, Wo3_ref[...]) + bo3_ref[...]
    pred_ref[...] = inp + p
    hidden_out_ref[...] = hidden_new


def kernel(inputs, hidden, edges, edge_index, W1, b1, W2, b2,
           Whr, Whi, Whh, Wir, bir, Wii, bii, Win, binn,
           Wo1, bo1, Wo2, bo2, Wo3, bo3):
    del edge_index
    f32 = jnp.float32
    edges_t = jnp.transpose(edges.reshape(_N, _N - 1, 4), (2, 0, 1))
    row2 = lambda b: b.reshape(1, -1).astype(f32)
    out = pl.pallas_call(
        _decoder_body,
        out_shape=(jax.ShapeDtypeStruct(inputs.shape, f32),
                   jax.ShapeDtypeStruct(hidden.shape, f32)),
        scratch_shapes=[
            pltpu.VMEM((_N, _F), jnp.bfloat16),
            pltpu.VMEM((_N, _F), jnp.bfloat16),
            pltpu.VMEM((_N, _N), jnp.bfloat16),
            pltpu.VMEM((_N, _F), f32),
            pltpu.VMEM((_N, _F), f32),
        ],
    )(edges_t, hidden, inputs,
      W1, row2(b1), W2, row2(b2),
      Whr, Whi, Whh,
      Wir, row2(bir), Wii, row2(bii), Win, row2(binn),
      Wo1, row2(bo1), Wo2, row2(bo2), Wo3, row2(bo3))
    return out


# speedup vs baseline: 1.9167x; 1.0088x over previous
import jax
import jax.numpy as jnp
from jax.experimental import pallas as pl
from jax.experimental.pallas import tpu as pltpu

_N = 256
_H = 256
_F = 2 * _H
_RB = 128

_DEFAULT = jax.lax.Precision.DEFAULT


def _mmT(x, w, precision=_DEFAULT):
    return jax.lax.dot_general(
        x, w, (((1,), (1,)), ((), ())),
        preferred_element_type=jnp.float32, precision=precision)


def _elu(x):
    return jnp.where(x > 0, x, jnp.exp(x) - 1.0)


def _decoder_body(edges_ref, hidden_ref, inputs_ref,
                  W1_ref, b1_ref, W2_ref, b2_ref,
                  Whr_ref, Whi_ref, Whh_ref,
                  Wir_ref, bir_ref, Wii_ref, bii_ref, Win_ref, binn_ref,
                  Wo1_ref, bo1_ref, Wo2_ref, bo2_ref, Wo3_ref, bo3_ref,
                  pred_ref, hidden_out_ref,
                  A_ref, B_ref, w_ref, S_ref, Q_ref):
    f32 = jnp.float32
    bf16 = jnp.bfloat16
    log2e = 1.4426950408889634
    ln2 = 0.6931471805599453
    hidden = hidden_ref[...]
    W1 = W1_ref[...]
    dflt = jax.lax.Precision.DEFAULT
    h2 = hidden * log2e
    A_ref[...] = _mmT(h2, W1[:, :_H], dflt).astype(bf16)
    B_ref[...] = _mmT(h2, W1[:, _H:], dflt).astype(bf16)

    e4 = edges_ref[...]
    wmax = jnp.maximum(jnp.maximum(e4[0], e4[1]), jnp.maximum(e4[2], e4[3]))
    zcol = jnp.zeros((_N, 1), f32)
    u = jnp.concatenate([wmax, zcol], axis=1)
    v = jnp.concatenate([zcol, wmax], axis=1)
    row = jax.lax.broadcasted_iota(jnp.int32, (_N, _N), 0)
    col = jax.lax.broadcasted_iota(jnp.int32, (_N, _N), 1)
    w_ref[...] = jnp.where(col < row, u, jnp.where(col > row, v, 0.0)
                           ).astype(bf16)

    b1 = b1_ref[...]
    b1_lo = (b1 * log2e).astype(bf16)
    S_ref[...] = jnp.zeros((_N, _F), f32)
    Q_ref[...] = jnp.zeros((_N, _F), f32)

    def _tree(t):
        s64 = t[0:64] + t[64:128]
        s32 = s64[0:32] + s64[32:64]
        s16 = s32[0:16] + s32[16:32]
        s8 = s16[0:8] + s16[8:16]
        s4 = s8[0:4] + s8[4:8]
        s2 = s4[0:2] + s4[2:4]
        return (s2[0] + s2[1]).astype(f32)

    def step(rb, carry):
        r0 = rb * _RB
        w_blk = w_ref[pl.ds(r0, _RB), :]
        b_blk = B_ref[pl.ds(r0, _RB), :]
        x = (w_blk[:, :, None] * (A_ref[...][None, :, :] + b_blk[:, None, :])
             + b1_lo[None, :, :])
        xn = jnp.minimum(x, 0)
        S_ref[...] += _tree(xn)
        Q_ref[...] += _tree(jnp.exp2(xn))
        return carry

    jax.lax.fori_loop(0, _N // _RB, step, 0)

    wmat = w_ref[...]
    sw = jnp.sum(wmat.astype(f32), axis=0).reshape(1, _N)
    wtB = jax.lax.dot_general(
        wmat, B_ref[...], (((0,), (0,)), ((), ())),
        preferred_element_type=f32)
    xlin = (A_ref[...].astype(f32) * sw.reshape(_N, 1)
            + wtB + _N * (b1 * log2e))
    S = (xlin - S_ref[...]) * ln2 + Q_ref[...] - (_N + _elu(b1))
    agg = _mmT(S, W2_ref[...]) * (1.0 / (_N - 1)) + b2_ref[...]

    inp = inputs_ref[...]
    r = jax.nn.sigmoid(_mmT(inp, Wir_ref[...]) + bir_ref[...]
                       + _mmT(agg, Whr_ref[...]))
    i = jax.nn.sigmoid(_mmT(inp, Wii_ref[...]) + bii_ref[...]
                       + _mmT(agg, Whi_ref[...]))
    n = jnp.tanh(_mmT(inp, Win_ref[...]) + binn_ref[...]
                 + r * _mmT(agg, Whh_ref[...]))
    hidden_new = (1.0 - i) * n + i * hidden

    p = jax.nn.relu(_mmT(hidden_new, Wo1_ref[...]) + bo1_ref[...])
    p = jax.nn.relu(_mmT(p, Wo2_ref[...]) + bo2_ref[...])
    p = _mmT(p, Wo3_ref[...]) + bo3_ref[...]
    pred_ref[...] = inp + p
    hidden_out_ref[...] = hidden_new


def kernel(inputs, hidden, edges, edge_index, W1, b1, W2, b2,
           Whr, Whi, Whh, Wir, bir, Wii, bii, Win, binn,
           Wo1, bo1, Wo2, bo2, Wo3, bo3):
    del edge_index
    f32 = jnp.float32
    edges_t = jnp.transpose(edges.reshape(_N, _N - 1, 4), (2, 0, 1))
    row2 = lambda b: b.reshape(1, -1).astype(f32)
    out = pl.pallas_call(
        _decoder_body,
        out_shape=(jax.ShapeDtypeStruct(inputs.shape, f32),
                   jax.ShapeDtypeStruct(hidden.shape, f32)),
        scratch_shapes=[
            pltpu.VMEM((_N, _F), jnp.bfloat16),
            pltpu.VMEM((_N, _F), jnp.bfloat16),
            pltpu.VMEM((_N, _N), jnp.bfloat16),
            pltpu.VMEM((_N, _F), f32),
            pltpu.VMEM((_N, _F), f32),
        ],
    )(edges_t, hidden, inputs,
      W1, row2(b1), W2, row2(b2),
      Whr, Whi, Whh,
      Wir, row2(bir), Wii, row2(bii), Win, row2(binn),
      Wo1, row2(bo1), Wo2, row2(bo2), Wo3, row2(bo3))
    return out
